# Initial kernel scaffold; baseline (speedup 1.0000x reference)
import jax
import jax.numpy as jnp
from jax.experimental import pallas as pl


def kernel(x, freqs_cis, Wq, Wk, Wv, Wo):
    raise NotImplementedError("write your pallas kernel here")


# speedup vs baseline: 6.6839x; 6.6839x over previous
import functools

import jax
import jax.numpy as jnp
import numpy as np
from jax.experimental import pallas as pl

_B, _N, _C = 2, 2048, 768
_H = 12
_Dh = _C // _H
_R = 256
_HALF = _N // 2
_NM = _N - _R


def _match_kernel(xe_ref, xo_ref, wk_ref, bs_ref, bb_ref):
    a = jnp.dot(xe_ref[0], wk_ref[...], preferred_element_type=jnp.float32)
    b = jnp.dot(xo_ref[0], wk_ref[...], preferred_element_type=jnp.float32)
    an = a / jnp.maximum(
        jnp.sqrt(jnp.sum(a * a, axis=1, keepdims=True)), 1e-12)
    bn = b / jnp.maximum(
        jnp.sqrt(jnp.sum(b * b, axis=1, keepdims=True)), 1e-12)
    scores = jax.lax.dot_general(
        an, bn, (((1,), (1,)), ((), ())),
        preferred_element_type=jnp.float32)
    m = jnp.max(scores, axis=1, keepdims=True)
    cols = jax.lax.broadcasted_iota(jnp.int32, scores.shape, 1)
    cand = jnp.where(scores == m, cols, jnp.int32(_HALF))
    bs_ref[0, 0, :] = m[:, 0]
    bb_ref[0, 0, :] = jnp.min(cand, axis=1)


def _match(x_even, x_odd, wk):
    bs, bb = pl.pallas_call(
        _match_kernel,
        grid=(_B,),
        in_specs=[
            pl.BlockSpec((1, _HALF, _C), lambda i: (i, 0, 0)),
            pl.BlockSpec((1, _HALF, _C), lambda i: (i, 0, 0)),
            pl.BlockSpec((_C, _C), lambda i: (0, 0)),
        ],
        out_specs=[
            pl.BlockSpec((1, 1, _HALF), lambda i: (i, 0, 0)),
            pl.BlockSpec((1, 1, _HALF), lambda i: (i, 0, 0)),
        ],
        out_shape=[
            jax.ShapeDtypeStruct((_B, 1, _HALF), jnp.float32),
            jax.ShapeDtypeStruct((_B, 1, _HALF), jnp.int32),
        ],
    )(x_even, x_odd, wk)
    return bs[:, 0, :], bb[:, 0, :]


_HP = 2
_HB = _H // _HP


def _qkv_kernel(xm_ref, wq_ref, wk_ref, wv_ref, cos_ref, sin_ref, s_ref,
                q_ref, k_ref, v_ref):
    x = xm_ref[0]
    cos = cos_ref[...]
    sin = sin_ref[...]
    s_mat = s_ref[...]

    q = jnp.dot(x, wq_ref[...], preferred_element_type=jnp.float32)
    q_rot = jnp.dot(q, s_mat, preferred_element_type=jnp.float32)
    qr = q * cos + q_rot * sin
    q_ref[0, 0] = qr[:, :_Dh]
    q_ref[0, 1] = qr[:, _Dh:]

    k = jnp.dot(x, wk_ref[...], preferred_element_type=jnp.float32)
    k_rot = jnp.dot(k, s_mat, preferred_element_type=jnp.float32)
    kr = k * cos + k_rot * sin
    k_ref[0, 0] = kr[:, :_Dh]
    k_ref[0, 1] = kr[:, _Dh:]

    v = jnp.dot(x, wv_ref[...], preferred_element_type=jnp.float32)
    v_ref[0, 0] = v[:, :_Dh]
    v_ref[0, 1] = v[:, _Dh:]


def _qkv(x_m, wq, wk, wv, cos_i, sin_i, s_mat):
    wcols = _HP * _Dh
    return pl.pallas_call(
        _qkv_kernel,
        grid=(_B, _HB),
        in_specs=[
            pl.BlockSpec((1, _NM, _C), lambda b, h: (b, 0, 0)),
            pl.BlockSpec((_C, wcols), lambda b, h: (0, h)),
            pl.BlockSpec((_C, wcols), lambda b, h: (0, h)),
            pl.BlockSpec((_C, wcols), lambda b, h: (0, h)),
            pl.BlockSpec((_NM, wcols), lambda b, h: (0, 0)),
            pl.BlockSpec((_NM, wcols), lambda b, h: (0, 0)),
            pl.BlockSpec((wcols, wcols), lambda b, h: (0, 0)),
        ],
        out_specs=[
            pl.BlockSpec((1, _HP, _NM, _Dh), lambda b, h: (b, h, 0, 0)),
            pl.BlockSpec((1, _HP, _NM, _Dh), lambda b, h: (b, h, 0, 0)),
            pl.BlockSpec((1, _HP, _NM, _Dh), lambda b, h: (b, h, 0, 0)),
        ],
        out_shape=[
            jax.ShapeDtypeStruct((_B, _H, _NM, _Dh), jnp.float32),
            jax.ShapeDtypeStruct((_B, _H, _NM, _Dh), jnp.float32),
            jax.ShapeDtypeStruct((_B, _H, _NM, _Dh), jnp.float32),
        ],
    )(x_m, wq, wk, wv, cos_i, sin_i, s_mat)


def _attn_kernel(q_ref, k_ref, v_ref, o_ref):
    q = q_ref[0, 0]
    k = k_ref[0, 0]
    v = v_ref[0, 0]
    s = jax.lax.dot_general(
        q, k, (((1,), (1,)), ((), ())),
        preferred_element_type=jnp.float32)
    s = s * jnp.float32(1.0 / np.sqrt(_Dh))
    rows = jax.lax.broadcasted_iota(jnp.int32, s.shape, 0)
    cols = jax.lax.broadcasted_iota(jnp.int32, s.shape, 1)
    s = jnp.where(rows >= cols, s, jnp.float32(-1e9))
    m = jnp.max(s, axis=1, keepdims=True)
    e = jnp.exp(s - m)
    p = e / jnp.sum(e, axis=1, keepdims=True)
    o_ref[0, 0] = jnp.dot(p, v, preferred_element_type=jnp.float32)


def _attention(q, k, v):
    return pl.pallas_call(
        _attn_kernel,
        grid=(_B, _H),
        in_specs=[
            pl.BlockSpec((1, 1, _NM, _Dh), lambda b, h: (b, h, 0, 0)),
            pl.BlockSpec((1, 1, _NM, _Dh), lambda b, h: (b, h, 0, 0)),
            pl.BlockSpec((1, 1, _NM, _Dh), lambda b, h: (b, h, 0, 0)),
        ],
        out_specs=pl.BlockSpec((1, 1, _NM, _Dh), lambda b, h: (b, h, 0, 0)),
        out_shape=jax.ShapeDtypeStruct((_B, _H, _NM, _Dh), jnp.float32),
    )(q, k, v)


def _proj_kernel(a_ref, wo_ref, o_ref):
    acc = jnp.zeros((_NM, _C), dtype=jnp.float32)
    for h in range(_H):
        acc = acc + jnp.dot(
            a_ref[0, h], wo_ref[h * _Dh:(h + 1) * _Dh, :],
            preferred_element_type=jnp.float32)
    o_ref[0] = acc


def _out_proj(att, wo):
    return pl.pallas_call(
        _proj_kernel,
        grid=(_B,),
        in_specs=[
            pl.BlockSpec((1, _H, _NM, _Dh), lambda b: (b, 0, 0, 0)),
            pl.BlockSpec((_C, _C), lambda b: (0, 0)),
        ],
        out_specs=pl.BlockSpec((1, _NM, _C), lambda b: (b, 0, 0)),
        out_shape=jax.ShapeDtypeStruct((_B, _NM, _C), jnp.float32),
    )(att, wo)


def _select_pairs(best_s, best_b):
    bi = jnp.arange(_B, dtype=jnp.int32)[:, None]
    order = jnp.argsort(-best_s, axis=1).astype(jnp.int32)
    bb_ord = jnp.take_along_axis(best_b, order, axis=1)
    pos = jnp.arange(_HALF, dtype=jnp.int32)[None, :]
    firstpos = jnp.full((_B, _HALF), _HALF, dtype=jnp.int32)
    firstpos = firstpos.at[bi, bb_ord].min(
        jnp.broadcast_to(pos, (_B, _HALF)))
    take0 = jnp.take_along_axis(firstpos, bb_ord, axis=1) == pos
    c0 = jnp.cumsum(take0.astype(jnp.int32), axis=1) - take0.astype(jnp.int32)
    take = jnp.logical_and(take0, c0 < _R)
    cnt = jnp.sum(take.astype(jnp.int32), axis=1)
    sel = jnp.argsort(jnp.logical_not(take).astype(jnp.int32),
                      axis=1).astype(jnp.int32)[:, :_R]
    ca = jnp.take_along_axis(order, sel, axis=1)
    slots = jnp.arange(_R, dtype=jnp.int32)[None, :]
    ca = jnp.where(slots < cnt[:, None], ca, ca[:, :1])
    cb = jnp.take_along_axis(best_b, ca, axis=1)
    ga = ca * 2
    gb = cb * 2 + 1
    rm = jnp.zeros((_B, _N), dtype=bool).at[bi, gb].set(True)
    keep = jnp.argsort(rm.astype(jnp.int32), axis=1).astype(jnp.int32)[:, :_NM]
    return ga, gb, keep, rm


def _build_s_mat():
    s = np.zeros((_HP * _Dh, _HP * _Dh), dtype=np.float32)
    for h in range(_HP):
        o = h * _Dh
        for i in range(_Dh // 2):
            s[o + 2 * i + 1, o + 2 * i] = -1.0
            s[o + 2 * i, o + 2 * i + 1] = 1.0
    return jnp.asarray(s)


@jax.jit
def _run(x, freqs_cis, wq, wk, wv, wo):
    bi = jnp.arange(_B, dtype=jnp.int32)[:, None]

    x_even = x[:, 0::2, :]
    x_odd = x[:, 1::2, :]
    best_s, best_b = _match(x_even, x_odd, wk)
    ga, gb, keep, rm = _select_pairs(best_s, best_b)

    pmerge = jnp.broadcast_to(
        jnp.arange(_N, dtype=jnp.int32)[None, :], (_B, _N))
    pmerge = pmerge.at[bi, ga].set(gb)
    idx2 = jnp.take_along_axis(pmerge, keep, axis=1)
    x1 = jnp.take_along_axis(x, keep[:, :, None], axis=1)
    x2 = jnp.take_along_axis(x, idx2[:, :, None], axis=1)
    x_m = (x1 + x2) * 0.5

    cos = freqs_cis[:_NM, :, 0]
    sin = freqs_cis[:_NM, :, 1]
    cos_i = jnp.tile(jnp.repeat(cos, 2, axis=1), (1, _HP))
    sin_i = jnp.tile(jnp.repeat(sin, 2, axis=1), (1, _HP))
    s_mat = _build_s_mat()

    q, k, v = _qkv(x_m, wq, wk, wv, cos_i, sin_i, s_mat)
    att = _attention(q, k, v)
    out_m = _out_proj(att, wo)

    rows = jnp.zeros((_B, _N), dtype=jnp.int32).at[bi, keep].set(
        jnp.broadcast_to(jnp.arange(_NM, dtype=jnp.int32)[None, :],
                         (_B, _NM)))
    in_keep = jnp.zeros((_B, _N), dtype=bool).at[bi, keep].set(True)
    pb = jnp.broadcast_to(
        jnp.arange(_N, dtype=jnp.int32)[None, :], (_B, _N))
    pb = pb.at[bi, gb].set(ga)
    src = jnp.where(rm, jnp.take_along_axis(rows, pb, axis=1), rows)
    valid = jnp.logical_or(
        in_keep,
        jnp.logical_and(rm, jnp.take_along_axis(in_keep, pb, axis=1)))
    out = jnp.where(
        valid[:, :, None],
        jnp.take_along_axis(out_m, src[:, :, None], axis=1),
        jnp.float32(0.0))
    return out, k, v


def kernel(x, freqs_cis, Wq, Wk, Wv, Wo):
    return _run(x, freqs_cis, Wq, Wk, Wv, Wo)
